# Initial kernel scaffold; baseline (speedup 1.0000x reference)
import jax
import jax.numpy as jnp
from jax.experimental import pallas as pl


def kernel(x, targets, centers):
    raise NotImplementedError("write your pallas kernel here")


# speedup vs baseline: 2.6204x; 2.6204x over previous
import functools

import jax
import jax.numpy as jnp
from jax.experimental import pallas as pl
from jax.experimental.pallas import tpu as pltpu

NBINS = 64
MAX_DIST = 2.0
TB = 512
CPAD = 1024


def _stage1(x_ref, c_ref, t_ref, pos_ref, neg_ref):
    xb = x_ref[...]
    cb = c_ref[...]
    tb = t_ref[...]
    xc = jax.lax.dot_general(
        xb, cb, (((1,), (1,)), ((), ())),
        preferred_element_type=jnp.float32)
    x2 = jnp.sum(xb * xb, axis=1, keepdims=True)
    c2 = jnp.sum(cb * cb, axis=1)[None, :]
    d2 = x2 + c2 - 2.0 * xc
    dist = jnp.sqrt(jnp.clip(d2, 1e-12, None))
    col = jax.lax.broadcasted_iota(jnp.int32, (TB, CPAD), 1)
    eq = col == tb
    pad = col >= 1000
    pos_ref[...] = jnp.sum(jnp.where(eq, dist, 0.0), axis=1, keepdims=True)
    neg_ref[...] = jnp.min(jnp.where(eq | pad, jnp.inf, dist),
                           axis=1, keepdims=True)


def _stage2(pos_ref, neg_ref, out_ref, hist_ref, cdf_ref):
    pos = pos_ref[...]
    neg = neg_ref[...]
    hv = pos - neg
    mx = jnp.maximum(jnp.max(hv), MAX_DIST)
    mn = jnp.minimum(jnp.min(hv), -MAX_DIST)
    bw = (mx - mn) / (NBINS - 1)
    lo_f = jnp.floor((hv - mn) / bw)
    alpha = 1.0 - (hv - mn - lo_f * bw) / bw
    one_m_alpha = 1.0 - alpha

    def hist_body(b, _):
        bf = b.astype(jnp.float32)
        h = jnp.sum(jnp.where(lo_f == bf, alpha, 0.0))
        h += jnp.sum(jnp.where(lo_f == bf - 1.0, one_m_alpha, 0.0))
        h += jnp.where(b == NBINS - 1,
                       jnp.sum(jnp.where(lo_f == bf, one_m_alpha, 0.0)), 0.0)
        hist_ref[b] = h
        return _

    jax.lax.fori_loop(0, NBINS, hist_body, 0, unroll=True)

    def cdf_body(b, acc):
        acc += hist_ref[b]
        cdf_ref[b] = acc
        return acc

    total = jax.lax.fori_loop(0, NBINS, cdf_body, 0.0, unroll=True)

    def w_body(b, acc):
        cb = cdf_ref[b] / total
        sel = lo_f == b.astype(jnp.float32)
        wp = jnp.sum(jnp.where(sel, pos, 0.0)) * cb
        wn = jnp.sum(jnp.where(sel, neg, 0.0)) * cb
        return acc + (wp - wn)

    acc = jax.lax.fori_loop(0, NBINS, w_body, 0.0, unroll=True)
    out_ref[...] = jnp.full((1, 1), acc / (128.0 * 128.0), jnp.float32)


def kernel(x, targets, centers):
    B, D = x.shape
    C = centers.shape[0]
    centers_p = jnp.zeros((CPAD, D), x.dtype).at[:C].set(centers)
    tgt = targets.astype(jnp.int32).reshape(B, 1)

    pos, neg = pl.pallas_call(
        _stage1,
        grid=(B // TB,),
        in_specs=[
            pl.BlockSpec((TB, D), lambda i: (i, 0)),
            pl.BlockSpec((CPAD, D), lambda i: (0, 0)),
            pl.BlockSpec((TB, 1), lambda i: (i, 0)),
        ],
        out_specs=[
            pl.BlockSpec((TB, 1), lambda i: (i, 0)),
            pl.BlockSpec((TB, 1), lambda i: (i, 0)),
        ],
        out_shape=[
            jax.ShapeDtypeStruct((B, 1), jnp.float32),
            jax.ShapeDtypeStruct((B, 1), jnp.float32),
        ],
    )(x, centers_p, tgt)

    loss = pl.pallas_call(
        _stage2,
        out_shape=jax.ShapeDtypeStruct((1, 1), jnp.float32),
        scratch_shapes=[
            pltpu.SMEM((NBINS,), jnp.float32),
            pltpu.SMEM((NBINS,), jnp.float32),
        ],
    )(pos.reshape(128, 128), neg.reshape(128, 128))
    return loss.reshape(())


# speedup vs baseline: 2.8112x; 1.0728x over previous
import functools

import jax
import jax.numpy as jnp
from jax.experimental import pallas as pl
from jax.experimental.pallas import tpu as pltpu

NBINS = 64
MAX_DIST = 2.0
TB = 1024
CPAD = 1024


def _stage1(x_ref, c_ref, t_ref, pos_ref, neg_ref, c2_ref):
    @pl.when(pl.program_id(0) == 0)
    def _():
        cb0 = c_ref[...]
        c2_ref[...] = jnp.sum(cb0 * cb0, axis=1)[None, :]

    xb = x_ref[...]
    cb = c_ref[...]
    tb = t_ref[...]
    xc = jax.lax.dot_general(
        xb, cb, (((1,), (1,)), ((), ())),
        preferred_element_type=jnp.float32)
    x2 = jnp.sum(xb * xb, axis=1, keepdims=True)
    d2 = x2 + c2_ref[...] - 2.0 * xc
    col = jax.lax.broadcasted_iota(jnp.int32, (TB, CPAD), 1)
    eq = col == tb
    pad = col >= 1000
    pos_ref[...] = jnp.sum(jnp.where(eq, d2, 0.0), axis=1, keepdims=True)
    neg_ref[...] = jnp.min(jnp.where(eq | pad, jnp.inf, d2),
                           axis=1, keepdims=True)


def _stage2(pos_ref, neg_ref, out_ref, hist_ref, cdf_ref):
    pos = jnp.sqrt(jnp.clip(pos_ref[...], 1e-12, None))
    neg = jnp.sqrt(jnp.clip(neg_ref[...], 1e-12, None))
    hv = pos - neg
    mx = jnp.maximum(jnp.max(hv), MAX_DIST)
    mn = jnp.minimum(jnp.min(hv), -MAX_DIST)
    bw = (mx - mn) / (NBINS - 1)
    lo_f = jnp.floor((hv - mn) / bw)
    alpha = 1.0 - (hv - mn - lo_f * bw) / bw
    one_m_alpha = 1.0 - alpha

    def hist_body(b, carry):
        bf = b.astype(jnp.float32)
        h = jnp.sum(jnp.where(lo_f == bf, alpha, 0.0))
        h += jnp.sum(jnp.where(lo_f == bf - 1.0, one_m_alpha, 0.0))
        h += jnp.where(b == NBINS - 1,
                       jnp.sum(jnp.where(lo_f == bf, one_m_alpha, 0.0)), 0.0)
        hist_ref[b] = h
        return carry

    jax.lax.fori_loop(0, NBINS, hist_body, 0)

    def cdf_body(b, acc):
        acc += hist_ref[b]
        cdf_ref[b] = acc
        return acc

    total = jax.lax.fori_loop(0, NBINS, cdf_body, 0.0)

    def w_body(b, acc):
        cw = cdf_ref[b] / total
        sel = lo_f == b.astype(jnp.float32)
        wp = jnp.sum(jnp.where(sel, pos, 0.0)) * cw
        wn = jnp.sum(jnp.where(sel, neg, 0.0)) * cw
        return acc + (wp - wn)

    acc = jax.lax.fori_loop(0, NBINS, w_body, 0.0)
    out_ref[...] = jnp.full((1, 1), acc / (128.0 * 128.0), jnp.float32)


def kernel(x, targets, centers):
    B, D = x.shape
    C = centers.shape[0]
    centers_p = jnp.zeros((CPAD, D), x.dtype).at[:C].set(centers)
    tgt = targets.astype(jnp.int32).reshape(B, 1)

    pos2, neg2 = pl.pallas_call(
        _stage1,
        grid=(B // TB,),
        in_specs=[
            pl.BlockSpec((TB, D), lambda i: (i, 0)),
            pl.BlockSpec((CPAD, D), lambda i: (0, 0)),
            pl.BlockSpec((TB, 1), lambda i: (i, 0)),
        ],
        out_specs=[
            pl.BlockSpec((TB, 1), lambda i: (i, 0)),
            pl.BlockSpec((TB, 1), lambda i: (i, 0)),
        ],
        out_shape=[
            jax.ShapeDtypeStruct((B, 1), jnp.float32),
            jax.ShapeDtypeStruct((B, 1), jnp.float32),
        ],
        scratch_shapes=[pltpu.VMEM((1, CPAD), jnp.float32)],
    )(x, centers_p, tgt)

    loss = pl.pallas_call(
        _stage2,
        out_shape=jax.ShapeDtypeStruct((1, 1), jnp.float32),
        scratch_shapes=[
            pltpu.SMEM((NBINS,), jnp.float32),
            pltpu.SMEM((NBINS,), jnp.float32),
        ],
    )(pos2.reshape(128, 128), neg2.reshape(128, 128))
    return loss.reshape(())


# speedup vs baseline: 3.4803x; 1.2380x over previous
import functools

import jax
import jax.numpy as jnp
from jax.experimental import pallas as pl
from jax.experimental.pallas import tpu as pltpu

NBINS = 64
MAX_DIST = 2.0
TB = 1024
CPAD = 1024


def _stage1(x_ref, c_ref, t_ref, pos_ref, neg_ref, c2_ref):
    @pl.when(pl.program_id(0) == 0)
    def _():
        cb0 = c_ref[...]
        c2_ref[...] = jnp.sum(cb0 * cb0, axis=1)[None, :]

    xb = x_ref[...]
    cb = c_ref[...]
    tb = t_ref[...]
    C = cb.shape[0]
    xc = jax.lax.dot_general(
        xb, cb, (((1,), (1,)), ((), ())),
        preferred_element_type=jnp.float32)
    x2 = jnp.sum(xb * xb, axis=1, keepdims=True)
    d2 = x2 + c2_ref[...] - 2.0 * xc
    col = jax.lax.broadcasted_iota(jnp.int32, (TB, C), 1)
    eq = col == tb
    pose = jnp.sum(jnp.where(eq, d2, 0.0), axis=1, keepdims=True)
    nege = jnp.min(jnp.where(eq, jnp.inf, d2), axis=1, keepdims=True)
    pos_ref[...] = pose.reshape(TB // 128, 128)
    neg_ref[...] = nege.reshape(TB // 128, 128)


def _stage2(pos_ref, neg_ref, out_ref, hist_ref, cdf_ref):
    pos = jnp.sqrt(jnp.clip(pos_ref[...], 1e-12, None))
    neg = jnp.sqrt(jnp.clip(neg_ref[...], 1e-12, None))
    hv = pos - neg
    mx = jnp.maximum(jnp.max(hv), MAX_DIST)
    mn = jnp.minimum(jnp.min(hv), -MAX_DIST)
    bw = (mx - mn) / (NBINS - 1)
    lo_f = jnp.floor((hv - mn) / bw)
    alpha = 1.0 - (hv - mn - lo_f * bw) / bw
    one_m_alpha = 1.0 - alpha

    def hist_body(b, carry):
        bf = b.astype(jnp.float32)
        h = jnp.sum(jnp.where(lo_f == bf, alpha, 0.0))
        h += jnp.sum(jnp.where(lo_f == bf - 1.0, one_m_alpha, 0.0))
        h += jnp.where(b == NBINS - 1,
                       jnp.sum(jnp.where(lo_f == bf, one_m_alpha, 0.0)), 0.0)
        hist_ref[b] = h
        return carry

    jax.lax.fori_loop(0, NBINS, hist_body, 0)

    def cdf_body(b, acc):
        acc += hist_ref[b]
        cdf_ref[b] = acc
        return acc

    total = jax.lax.fori_loop(0, NBINS, cdf_body, 0.0)

    def w_body(b, acc):
        cw = cdf_ref[b] / total
        sel = lo_f == b.astype(jnp.float32)
        wp = jnp.sum(jnp.where(sel, pos, 0.0)) * cw
        wn = jnp.sum(jnp.where(sel, neg, 0.0)) * cw
        return acc + (wp - wn)

    acc = jax.lax.fori_loop(0, NBINS, w_body, 0.0)
    out_ref[...] = jnp.full((1, 1), acc / (128.0 * 128.0), jnp.float32)


def kernel(x, targets, centers):
    B, D = x.shape
    C = centers.shape[0]
    tgt = targets.astype(jnp.int32).reshape(B, 1)
    R = TB // 128

    pos2, neg2 = pl.pallas_call(
        _stage1,
        grid=(B // TB,),
        in_specs=[
            pl.BlockSpec((TB, D), lambda i: (i, 0)),
            pl.BlockSpec((C, D), lambda i: (0, 0)),
            pl.BlockSpec((TB, 1), lambda i: (i, 0)),
        ],
        out_specs=[
            pl.BlockSpec((R, 128), lambda i: (i, 0)),
            pl.BlockSpec((R, 128), lambda i: (i, 0)),
        ],
        out_shape=[
            jax.ShapeDtypeStruct((B // 128, 128), jnp.float32),
            jax.ShapeDtypeStruct((B // 128, 128), jnp.float32),
        ],
        scratch_shapes=[pltpu.VMEM((1, C), jnp.float32)],
    )(x, centers, tgt)

    loss = pl.pallas_call(
        _stage2,
        out_shape=jax.ShapeDtypeStruct((1, 1), jnp.float32),
        scratch_shapes=[
            pltpu.SMEM((NBINS,), jnp.float32),
            pltpu.SMEM((NBINS,), jnp.float32),
        ],
    )(pos2, neg2)
    return loss.reshape(())
